# baseline (speedup 1.0000x reference)
import functools

import jax
import jax.numpy as jnp
from jax.experimental import pallas as pl
from jax.experimental.pallas import tpu as pltpu

_N = 10000
_BM1 = 512
_BR = 256


def _pass1_kernel(adj_ref, x_ref, W1_ref, b1_ref, W2_ref, Wf_ref,
                  z2_ref, z1_s, w2f_s):
    @pl.when(pl.program_id(0) == 0)
    def _():
        z1_s[...] = jnp.dot(x_ref[...], W1_ref[...],
                            preferred_element_type=jnp.float32
                            ).astype(jnp.bfloat16)
        w2f_s[...] = jnp.dot(W2_ref[...], Wf_ref[...],
                             preferred_element_type=jnp.float32)
    ab = adj_ref[...].astype(jnp.bfloat16)
    t = jnp.dot(ab, z1_s[...], preferred_element_type=jnp.float32)
    h = jnp.maximum(t + b1_ref[...], 0.0)
    z2_ref[...] = jnp.dot(h, w2f_s[...], preferred_element_type=jnp.float32)


def _pass2_kernel(ti_ref, adj_hbm, z2_ref, b2_ref, Wf_ref, bf_ref,
                  out_ref, gath_s, sem):
    step = pl.program_id(0)
    nsteps = pl.num_programs(0)

    def issue(buf, base):
        def body(r, carry):
            i0 = base + 8 * r
            pltpu.make_async_copy(adj_hbm.at[ti_ref[i0 + 0]],
                                  gath_s.at[buf, 8 * r + 0], sem.at[buf, 0]).start()
            pltpu.make_async_copy(adj_hbm.at[ti_ref[i0 + 1]],
                                  gath_s.at[buf, 8 * r + 1], sem.at[buf, 1]).start()
            pltpu.make_async_copy(adj_hbm.at[ti_ref[i0 + 2]],
                                  gath_s.at[buf, 8 * r + 2], sem.at[buf, 2]).start()
            pltpu.make_async_copy(adj_hbm.at[ti_ref[i0 + 3]],
                                  gath_s.at[buf, 8 * r + 3], sem.at[buf, 3]).start()
            pltpu.make_async_copy(adj_hbm.at[ti_ref[i0 + 4]],
                                  gath_s.at[buf, 8 * r + 4], sem.at[buf, 4]).start()
            pltpu.make_async_copy(adj_hbm.at[ti_ref[i0 + 5]],
                                  gath_s.at[buf, 8 * r + 5], sem.at[buf, 5]).start()
            pltpu.make_async_copy(adj_hbm.at[ti_ref[i0 + 6]],
                                  gath_s.at[buf, 8 * r + 6], sem.at[buf, 6]).start()
            pltpu.make_async_copy(adj_hbm.at[ti_ref[i0 + 7]],
                                  gath_s.at[buf, 8 * r + 7], sem.at[buf, 7]).start()
            return carry
        jax.lax.fori_loop(0, _BR // 8, body, 0)

    @pl.when(step == 0)
    def _():
        issue(0, 0)

    @pl.when(step + 1 < nsteps)
    def _():
        issue((step + 1) % 2, (step + 1) * _BR)

    cur = step % 2

    def drain(r, carry):
        pltpu.make_async_copy(adj_hbm.at[0], gath_s.at[cur, 0],
                              sem.at[cur, 0]).wait()
        pltpu.make_async_copy(adj_hbm.at[0], gath_s.at[cur, 1],
                              sem.at[cur, 1]).wait()
        pltpu.make_async_copy(adj_hbm.at[0], gath_s.at[cur, 2],
                              sem.at[cur, 2]).wait()
        pltpu.make_async_copy(adj_hbm.at[0], gath_s.at[cur, 3],
                              sem.at[cur, 3]).wait()
        pltpu.make_async_copy(adj_hbm.at[0], gath_s.at[cur, 4],
                              sem.at[cur, 4]).wait()
        pltpu.make_async_copy(adj_hbm.at[0], gath_s.at[cur, 5],
                              sem.at[cur, 5]).wait()
        pltpu.make_async_copy(adj_hbm.at[0], gath_s.at[cur, 6],
                              sem.at[cur, 6]).wait()
        pltpu.make_async_copy(adj_hbm.at[0], gath_s.at[cur, 7],
                              sem.at[cur, 7]).wait()
        return carry

    jax.lax.fori_loop(0, _BR // 8, drain, 0)

    acc = jnp.dot(gath_s[cur], z2_ref[...], preferred_element_type=jnp.float32)
    bias = jnp.dot(b2_ref[...], Wf_ref[...],
                   preferred_element_type=jnp.float32) + bf_ref[...]
    o = acc + bias
    m = jnp.max(o, axis=1, keepdims=True)
    lse = m + jnp.log(jnp.sum(jnp.exp(o - m), axis=1, keepdims=True))
    out_ref[...] = o - lse


def kernel(x, adj, type_index, non_zero_index, non_zero_value,
           W1, b1, W2, b2, Wf, bf):
    n, nfeat = x.shape
    nhid2 = W1.shape[1]
    nhid = W2.shape[1]
    ncls = Wf.shape[1]
    t = type_index.shape[0]

    b1r = b1.reshape(1, nhid2)
    b2r = b2.reshape(1, nhid)
    bfr = bf.reshape(1, ncls)

    z2 = pl.pallas_call(
        _pass1_kernel,
        grid=(pl.cdiv(n, _BM1),),
        in_specs=[
            pl.BlockSpec((_BM1, n), lambda i: (i, 0)),
            pl.BlockSpec((n, nfeat), lambda i: (0, 0)),
            pl.BlockSpec((nfeat, nhid2), lambda i: (0, 0)),
            pl.BlockSpec((1, nhid2), lambda i: (0, 0)),
            pl.BlockSpec((nhid2, nhid), lambda i: (0, 0)),
            pl.BlockSpec((nhid, ncls), lambda i: (0, 0)),
        ],
        out_specs=pl.BlockSpec((_BM1, ncls), lambda i: (i, 0)),
        out_shape=jax.ShapeDtypeStruct((n, ncls), jnp.float32),
        scratch_shapes=[pltpu.VMEM((n, nhid2), jnp.bfloat16),
                        pltpu.VMEM((nhid2, ncls), jnp.float32)],
    )(adj, x, W1, b1r, W2, Wf)

    grid_spec = pltpu.PrefetchScalarGridSpec(
        num_scalar_prefetch=1,
        grid=(t // _BR,),
        in_specs=[
            pl.BlockSpec(memory_space=pl.ANY),
            pl.BlockSpec((n, ncls), lambda i, ti: (0, 0)),
            pl.BlockSpec((1, nhid), lambda i, ti: (0, 0)),
            pl.BlockSpec((nhid, ncls), lambda i, ti: (0, 0)),
            pl.BlockSpec((1, ncls), lambda i, ti: (0, 0)),
        ],
        out_specs=pl.BlockSpec((_BR, ncls), lambda i, ti: (i, 0)),
        scratch_shapes=[pltpu.VMEM((2, _BR, n), jnp.float32),
                        pltpu.SemaphoreType.DMA((2, 8))],
    )
    out = pl.pallas_call(
        _pass2_kernel,
        grid_spec=grid_spec,
        out_shape=jax.ShapeDtypeStruct((t, ncls), jnp.float32),
    )(type_index, adj, z2, b2r, Wf, bfr)
    return out


# speedup vs baseline: 1.0222x; 1.0222x over previous
import functools

import jax
import jax.numpy as jnp
from jax.experimental import pallas as pl
from jax.experimental.pallas import tpu as pltpu

_N = 10000
_BM1 = 512
_BR = 256


def _pass1_kernel(adj_ref, x_ref, W1_ref, b1_ref, W2_ref, Wf_ref,
                  z2_ref, z1_s, w2f_s):
    @pl.when(pl.program_id(0) == 0)
    def _():
        z1_s[...] = jnp.dot(x_ref[...], W1_ref[...],
                            preferred_element_type=jnp.float32
                            ).astype(jnp.bfloat16)
        w2f_s[...] = jnp.dot(W2_ref[...], Wf_ref[...],
                             preferred_element_type=jnp.float32)
    ab = adj_ref[...].astype(jnp.bfloat16)
    t = jnp.dot(ab, z1_s[...], preferred_element_type=jnp.float32)
    h = jnp.maximum(t + b1_ref[...], 0.0)
    z2_ref[...] = jnp.dot(h, w2f_s[...], preferred_element_type=jnp.float32)


def _pass2_kernel(ti_ref, adj_hbm, z2_ref, b2_ref, Wf_ref, bf_ref,
                  out_ref, gath_s, sem):
    step = pl.program_id(0)
    nsteps = pl.num_programs(0)

    def issue(buf, base):
        def body(r, carry):
            i0 = base + 8 * r
            pltpu.make_async_copy(adj_hbm.at[ti_ref[i0 + 0]],
                                  gath_s.at[buf, 8 * r + 0], sem.at[buf, 0]).start()
            pltpu.make_async_copy(adj_hbm.at[ti_ref[i0 + 1]],
                                  gath_s.at[buf, 8 * r + 1], sem.at[buf, 1]).start()
            pltpu.make_async_copy(adj_hbm.at[ti_ref[i0 + 2]],
                                  gath_s.at[buf, 8 * r + 2], sem.at[buf, 2]).start()
            pltpu.make_async_copy(adj_hbm.at[ti_ref[i0 + 3]],
                                  gath_s.at[buf, 8 * r + 3], sem.at[buf, 3]).start()
            pltpu.make_async_copy(adj_hbm.at[ti_ref[i0 + 4]],
                                  gath_s.at[buf, 8 * r + 4], sem.at[buf, 4]).start()
            pltpu.make_async_copy(adj_hbm.at[ti_ref[i0 + 5]],
                                  gath_s.at[buf, 8 * r + 5], sem.at[buf, 5]).start()
            pltpu.make_async_copy(adj_hbm.at[ti_ref[i0 + 6]],
                                  gath_s.at[buf, 8 * r + 6], sem.at[buf, 6]).start()
            pltpu.make_async_copy(adj_hbm.at[ti_ref[i0 + 7]],
                                  gath_s.at[buf, 8 * r + 7], sem.at[buf, 7]).start()
            return carry
        jax.lax.fori_loop(0, _BR // 8, body, 0, unroll=4)

    @pl.when(step == 0)
    def _():
        issue(0, 0)
        issue(1, _BR)

    @pl.when((step > 0) & (step + 1 < nsteps) & (step % 2 == 1))
    def _():
        issue(0, (step + 1) * _BR)

    @pl.when((step > 0) & (step + 1 < nsteps) & (step % 2 == 0))
    def _():
        issue(1, (step + 1) * _BR)

    cur = step % 2

    def drain(buf):
        def body(r, carry):
            pltpu.make_async_copy(adj_hbm.at[0], gath_s.at[buf, 0],
                                  sem.at[buf, 0]).wait()
            pltpu.make_async_copy(adj_hbm.at[0], gath_s.at[buf, 1],
                                  sem.at[buf, 1]).wait()
            pltpu.make_async_copy(adj_hbm.at[0], gath_s.at[buf, 2],
                                  sem.at[buf, 2]).wait()
            pltpu.make_async_copy(adj_hbm.at[0], gath_s.at[buf, 3],
                                  sem.at[buf, 3]).wait()
            pltpu.make_async_copy(adj_hbm.at[0], gath_s.at[buf, 4],
                                  sem.at[buf, 4]).wait()
            pltpu.make_async_copy(adj_hbm.at[0], gath_s.at[buf, 5],
                                  sem.at[buf, 5]).wait()
            pltpu.make_async_copy(adj_hbm.at[0], gath_s.at[buf, 6],
                                  sem.at[buf, 6]).wait()
            pltpu.make_async_copy(adj_hbm.at[0], gath_s.at[buf, 7],
                                  sem.at[buf, 7]).wait()
            return carry
        jax.lax.fori_loop(0, _BR // 8, body, 0, unroll=4)

    def finish(buf):
        drain(buf)
        acc = jnp.dot(gath_s[buf], z2_ref[...],
                      preferred_element_type=jnp.float32)
        bias = jnp.dot(b2_ref[...], Wf_ref[...],
                       preferred_element_type=jnp.float32) + bf_ref[...]
        o = acc + bias
        m = jnp.max(o, axis=1, keepdims=True)
        lse = m + jnp.log(jnp.sum(jnp.exp(o - m), axis=1, keepdims=True))
        out_ref[...] = o - lse

    @pl.when(cur == 0)
    def _():
        finish(0)

    @pl.when(cur == 1)
    def _():
        finish(1)


def kernel(x, adj, type_index, non_zero_index, non_zero_value,
           W1, b1, W2, b2, Wf, bf):
    n, nfeat = x.shape
    nhid2 = W1.shape[1]
    nhid = W2.shape[1]
    ncls = Wf.shape[1]
    t = type_index.shape[0]

    b1r = b1.reshape(1, nhid2)
    b2r = b2.reshape(1, nhid)
    bfr = bf.reshape(1, ncls)

    z2 = pl.pallas_call(
        _pass1_kernel,
        grid=(pl.cdiv(n, _BM1),),
        in_specs=[
            pl.BlockSpec((_BM1, n), lambda i: (i, 0)),
            pl.BlockSpec((n, nfeat), lambda i: (0, 0)),
            pl.BlockSpec((nfeat, nhid2), lambda i: (0, 0)),
            pl.BlockSpec((1, nhid2), lambda i: (0, 0)),
            pl.BlockSpec((nhid2, nhid), lambda i: (0, 0)),
            pl.BlockSpec((nhid, ncls), lambda i: (0, 0)),
        ],
        out_specs=pl.BlockSpec((_BM1, ncls), lambda i: (i, 0)),
        out_shape=jax.ShapeDtypeStruct((n, ncls), jnp.float32),
        scratch_shapes=[pltpu.VMEM((n, nhid2), jnp.bfloat16),
                        pltpu.VMEM((nhid2, ncls), jnp.float32)],
    )(adj, x, W1, b1r, W2, Wf)

    grid_spec = pltpu.PrefetchScalarGridSpec(
        num_scalar_prefetch=1,
        grid=(t // _BR,),
        in_specs=[
            pl.BlockSpec(memory_space=pl.ANY),
            pl.BlockSpec((n, ncls), lambda i, ti: (0, 0)),
            pl.BlockSpec((1, nhid), lambda i, ti: (0, 0)),
            pl.BlockSpec((nhid, ncls), lambda i, ti: (0, 0)),
            pl.BlockSpec((1, ncls), lambda i, ti: (0, 0)),
        ],
        out_specs=pl.BlockSpec((_BR, ncls), lambda i, ti: (i, 0)),
        scratch_shapes=[pltpu.VMEM((2, _BR, n), jnp.float32),
                        pltpu.SemaphoreType.DMA((2, 8))],
    )
    out = pl.pallas_call(
        _pass2_kernel,
        grid_spec=grid_spec,
        out_shape=jax.ShapeDtypeStruct((t, ncls), jnp.float32),
    )(type_index, adj, z2, b2r, Wf, bfr)
    return out
